# baseline (speedup 1.0000x reference)
import functools

import jax
import jax.numpy as jnp
from jax import lax
from jax.experimental import pallas as pl
from jax.experimental.pallas import tpu as pltpu

MARGIN_ = 1.0
EPS_ = 1e-6
CB_ = 1024
KAUG_ = 128
NBUF_ = 3
GW_ = 8


def _dist_body(a_aug_ref, labels_ref, target_hbm, minacc_ref, t_buf, sem,
               *, n_valid, nblocks):
    i = pl.program_id(0)
    B = a_aug_ref.shape[0]
    D = t_buf.shape[2]
    slot = lax.rem(i, NBUF_)

    def _start(idx):
        return jnp.where(idx == nblocks - 1, n_valid - CB_, idx * CB_)

    def _copy(idx, sl):
        return pltpu.make_async_copy(
            target_hbm.at[pl.ds(_start(idx), CB_)], t_buf.at[sl], sem.at[sl])

    @pl.when(i == 0)
    def _prime():
        _copy(0, 0).start()
        _copy(1, 1).start()

    @pl.when(i + 2 < nblocks)
    def _prefetch():
        _copy(i + 2, lax.rem(i + 2, NBUF_)).start()

    _copy(i, slot).wait()

    t = t_buf[slot] - EPS_
    t_sq = jnp.sum(t * t, axis=1, keepdims=True)
    t_aug = jnp.concatenate(
        [t * -2.0, t_sq, jnp.zeros((CB_, KAUG_ - D - 1), jnp.float32)],
        axis=1)

    s = lax.dot_general(a_aug_ref[...], t_aug, (((1,), (1,)), ((), ())),
                        preferred_element_type=jnp.float32)

    @pl.when(i == 0)
    def _init():
        minacc_ref[...] = jnp.full_like(minacc_ref, jnp.inf)

    lbl_s = labels_ref[...] - _start(i)
    lane = lax.broadcasted_iota(jnp.int32, (B, 128), 1)
    m = minacc_ref[...]
    for k in range(CB_ // 128):
        sk = s[:, k * 128:(k + 1) * 128]
        own = (lane + k * 128) == lbl_s
        m = jnp.minimum(m, jnp.where(own, jnp.inf, sk))
    minacc_ref[...] = m


def _final_body(labels_smem, minacc_ref, inputs_ref, target_hbm, out_ref,
                posbuf, sem):
    B = inputs_ref.shape[0]

    def _row_copy(b):
        return pltpu.make_async_copy(
            target_hbm.at[pl.ds(labels_smem[b], 1)],
            posbuf.at[pl.ds(b, 1)], sem)

    def _issue(b, carry):
        _row_copy(b).start()

        @pl.when(b >= GW_)
        def _drain_old():
            _row_copy(b - GW_).wait()
        return carry

    lax.fori_loop(0, B, _issue, 0)

    def _drain(b, carry):
        _row_copy(b).wait()
        return carry

    lax.fori_loop(B - GW_, B, _drain, 0)

    a = inputs_ref[...]
    a_sq = jnp.sum(a * a, axis=1, keepdims=True)
    d_an = jnp.sqrt(jnp.clip(
        a_sq + jnp.min(minacc_ref[...], axis=1, keepdims=True), 1e-12))
    dp = a - posbuf[...] + EPS_
    d_ap = jnp.sqrt(jnp.clip(jnp.sum(dp * dp, axis=1, keepdims=True), 1e-12))
    per = jnp.maximum(d_ap - d_an + MARGIN_, 0.0)
    out_ref[0, 0] = jnp.sum(per) / B


def kernel(inputs, labels, target):
    B, D = inputs.shape
    C = target.shape[0]
    nblocks = (C + CB_ - 1) // CB_

    a_aug = jnp.concatenate(
        [inputs,
         jnp.ones((B, 1), jnp.float32),
         jnp.zeros((B, KAUG_ - D - 1), jnp.float32)], axis=1)
    labels2 = labels.reshape(B, 1)

    minacc = pl.pallas_call(
        functools.partial(_dist_body, n_valid=C, nblocks=nblocks),
        grid=(nblocks,),
        in_specs=[
            pl.BlockSpec((B, KAUG_), lambda i: (0, 0)),
            pl.BlockSpec((B, 1), lambda i: (0, 0)),
            pl.BlockSpec(memory_space=pl.ANY),
        ],
        out_specs=pl.BlockSpec((B, 128), lambda i: (0, 0)),
        out_shape=jax.ShapeDtypeStruct((B, 128), jnp.float32),
        scratch_shapes=[
            pltpu.VMEM((NBUF_, CB_, D), jnp.float32),
            pltpu.SemaphoreType.DMA((NBUF_,)),
        ],
        compiler_params=pltpu.CompilerParams(
            dimension_semantics=("arbitrary",)),
    )(a_aug, labels2, target)

    out = pl.pallas_call(
        _final_body,
        in_specs=[
            pl.BlockSpec(memory_space=pltpu.SMEM),
            pl.BlockSpec((B, 128), lambda: (0, 0)),
            pl.BlockSpec((B, D), lambda: (0, 0)),
            pl.BlockSpec(memory_space=pl.ANY),
        ],
        out_specs=pl.BlockSpec(memory_space=pltpu.SMEM),
        out_shape=jax.ShapeDtypeStruct((1, 1), jnp.float32),
        scratch_shapes=[
            pltpu.VMEM((B, D), jnp.float32),
            pltpu.SemaphoreType.DMA,
        ],
    )(labels, minacc, inputs, target)
    return out[0, 0]


# speedup vs baseline: 1.3187x; 1.3187x over previous
import functools

import jax
import jax.numpy as jnp
from jax import lax
from jax.experimental import pallas as pl
from jax.experimental.pallas import tpu as pltpu

MARGIN_ = 1.0
EPS_ = 1e-6
CB_ = 1024
KAUG_ = 128
NBUF_ = 3
GW_ = 64


def _dist_body(a_aug_ref, labels_ref, target_hbm, minacc_ref, t_buf, sem,
               *, n_valid, nblocks):
    i = pl.program_id(0)
    B = a_aug_ref.shape[0]
    D = t_buf.shape[2]
    slot = lax.rem(i, NBUF_)

    def _start(idx):
        return jnp.where(idx == nblocks - 1, n_valid - CB_, idx * CB_)

    def _copy(idx, sl):
        return pltpu.make_async_copy(
            target_hbm.at[pl.ds(_start(idx), CB_)], t_buf.at[sl], sem.at[sl])

    @pl.when(i == 0)
    def _prime():
        _copy(0, 0).start()
        _copy(1, 1).start()

    @pl.when(i + 2 < nblocks)
    def _prefetch():
        _copy(i + 2, lax.rem(i + 2, NBUF_)).start()

    _copy(i, slot).wait()

    t = t_buf[slot] - EPS_
    t_sq = jnp.sum(t * t, axis=1, keepdims=True)
    t_aug = jnp.concatenate(
        [t * -2.0, t_sq, jnp.zeros((CB_, KAUG_ - D - 1), jnp.float32)],
        axis=1)

    s = lax.dot_general(a_aug_ref[...], t_aug, (((1,), (1,)), ((), ())),
                        preferred_element_type=jnp.float32)

    @pl.when(i == 0)
    def _init():
        minacc_ref[...] = jnp.full_like(minacc_ref, jnp.inf)

    lbl_s = labels_ref[...] - _start(i)
    lane = lax.broadcasted_iota(jnp.int32, (B, 128), 1)
    m = minacc_ref[...]
    for k in range(CB_ // 128):
        sk = s[:, k * 128:(k + 1) * 128]
        own = (lane + k * 128) == lbl_s
        m = jnp.minimum(m, jnp.where(own, jnp.inf, sk))
    minacc_ref[...] = m


def _final_body(labels_smem, minacc_ref, inputs_ref, target_hbm, out_ref,
                posbuf, sem):
    B = inputs_ref.shape[0]

    def _row_copy(b):
        return pltpu.make_async_copy(
            target_hbm.at[pl.ds(labels_smem[b], 1)],
            posbuf.at[pl.ds(b, 1)], sem)

    def _issue(b, carry):
        _row_copy(b).start()

        @pl.when(b >= GW_)
        def _drain_old():
            _row_copy(b - GW_).wait()
        return carry

    lax.fori_loop(0, B, _issue, 0)

    def _drain(b, carry):
        _row_copy(b).wait()
        return carry

    lax.fori_loop(B - GW_, B, _drain, 0)

    a = inputs_ref[...]
    a_sq = jnp.sum(a * a, axis=1, keepdims=True)
    d_an = jnp.sqrt(jnp.clip(
        a_sq + jnp.min(minacc_ref[...], axis=1, keepdims=True), 1e-12))
    dp = a - posbuf[...] + EPS_
    d_ap = jnp.sqrt(jnp.clip(jnp.sum(dp * dp, axis=1, keepdims=True), 1e-12))
    per = jnp.maximum(d_ap - d_an + MARGIN_, 0.0)
    out_ref[0, 0] = jnp.sum(per) / B


def kernel(inputs, labels, target):
    B, D = inputs.shape
    C = target.shape[0]
    nblocks = (C + CB_ - 1) // CB_

    a_aug = jnp.concatenate(
        [inputs,
         jnp.ones((B, 1), jnp.float32),
         jnp.zeros((B, KAUG_ - D - 1), jnp.float32)], axis=1)
    labels2 = labels.reshape(B, 1)

    minacc = pl.pallas_call(
        functools.partial(_dist_body, n_valid=C, nblocks=nblocks),
        grid=(nblocks,),
        in_specs=[
            pl.BlockSpec((B, KAUG_), lambda i: (0, 0)),
            pl.BlockSpec((B, 1), lambda i: (0, 0)),
            pl.BlockSpec(memory_space=pl.ANY),
        ],
        out_specs=pl.BlockSpec((B, 128), lambda i: (0, 0)),
        out_shape=jax.ShapeDtypeStruct((B, 128), jnp.float32),
        scratch_shapes=[
            pltpu.VMEM((NBUF_, CB_, D), jnp.float32),
            pltpu.SemaphoreType.DMA((NBUF_,)),
        ],
        compiler_params=pltpu.CompilerParams(
            dimension_semantics=("arbitrary",)),
    )(a_aug, labels2, target)

    out = pl.pallas_call(
        _final_body,
        in_specs=[
            pl.BlockSpec(memory_space=pltpu.SMEM),
            pl.BlockSpec((B, 128), lambda: (0, 0)),
            pl.BlockSpec((B, D), lambda: (0, 0)),
            pl.BlockSpec(memory_space=pl.ANY),
        ],
        out_specs=pl.BlockSpec(memory_space=pltpu.SMEM),
        out_shape=jax.ShapeDtypeStruct((1, 1), jnp.float32),
        scratch_shapes=[
            pltpu.VMEM((B, D), jnp.float32),
            pltpu.SemaphoreType.DMA,
        ],
    )(labels, minacc, inputs, target)
    return out[0, 0]


# speedup vs baseline: 1.3202x; 1.0012x over previous
import functools

import jax
import jax.numpy as jnp
from jax import lax
from jax.experimental import pallas as pl
from jax.experimental.pallas import tpu as pltpu

MARGIN_ = 1.0
EPS_ = 1e-6
CB_ = 1024
KAUG_ = 128
NBUF_ = 3
GW_ = 128


def _dist_body(a_aug_ref, labels_ref, target_hbm, minacc_ref, t_buf, sem,
               *, n_valid, nblocks):
    i = pl.program_id(0)
    B = a_aug_ref.shape[0]
    D = t_buf.shape[2]
    slot = lax.rem(i, NBUF_)

    def _start(idx):
        return jnp.where(idx == nblocks - 1, n_valid - CB_, idx * CB_)

    def _copy(idx, sl):
        return pltpu.make_async_copy(
            target_hbm.at[pl.ds(_start(idx), CB_)], t_buf.at[sl], sem.at[sl])

    @pl.when(i == 0)
    def _prime():
        _copy(0, 0).start()
        _copy(1, 1).start()

    @pl.when(i + 2 < nblocks)
    def _prefetch():
        _copy(i + 2, lax.rem(i + 2, NBUF_)).start()

    _copy(i, slot).wait()

    t = t_buf[slot] - EPS_
    t_sq = jnp.sum(t * t, axis=1, keepdims=True)
    t_aug = jnp.concatenate(
        [t * -2.0, t_sq, jnp.zeros((CB_, KAUG_ - D - 1), jnp.float32)],
        axis=1)

    s = lax.dot_general(a_aug_ref[...], t_aug, (((1,), (1,)), ((), ())),
                        preferred_element_type=jnp.float32)

    @pl.when(i == 0)
    def _init():
        minacc_ref[...] = jnp.full_like(minacc_ref, jnp.inf)

    lbl_s = labels_ref[...] - _start(i)
    lane = lax.broadcasted_iota(jnp.int32, (B, 128), 1)
    m = minacc_ref[...]
    for k in range(CB_ // 128):
        sk = s[:, k * 128:(k + 1) * 128]
        own = (lane + k * 128) == lbl_s
        m = jnp.minimum(m, jnp.where(own, jnp.inf, sk))
    minacc_ref[...] = m


def _final_body(labels_smem, minacc_ref, inputs_ref, target_hbm, out_ref,
                posbuf, sem):
    B = inputs_ref.shape[0]

    def _row_copy(b):
        return pltpu.make_async_copy(
            target_hbm.at[pl.ds(labels_smem[b], 1)],
            posbuf.at[pl.ds(b, 1)], sem)

    def _issue(b, carry):
        _row_copy(b).start()

        @pl.when(b >= GW_)
        def _drain_old():
            _row_copy(b - GW_).wait()
        return carry

    lax.fori_loop(0, B, _issue, 0)

    def _drain(b, carry):
        _row_copy(b).wait()
        return carry

    lax.fori_loop(B - GW_, B, _drain, 0)

    a = inputs_ref[...]
    a_sq = jnp.sum(a * a, axis=1, keepdims=True)
    d_an = jnp.sqrt(jnp.clip(
        a_sq + jnp.min(minacc_ref[...], axis=1, keepdims=True), 1e-12))
    dp = a - posbuf[...] + EPS_
    d_ap = jnp.sqrt(jnp.clip(jnp.sum(dp * dp, axis=1, keepdims=True), 1e-12))
    per = jnp.maximum(d_ap - d_an + MARGIN_, 0.0)
    out_ref[0, 0] = jnp.sum(per) / B


def kernel(inputs, labels, target):
    B, D = inputs.shape
    C = target.shape[0]
    nblocks = (C + CB_ - 1) // CB_

    a_aug = jnp.concatenate(
        [inputs,
         jnp.ones((B, 1), jnp.float32),
         jnp.zeros((B, KAUG_ - D - 1), jnp.float32)], axis=1)
    labels2 = labels.reshape(B, 1)

    minacc = pl.pallas_call(
        functools.partial(_dist_body, n_valid=C, nblocks=nblocks),
        grid=(nblocks,),
        in_specs=[
            pl.BlockSpec((B, KAUG_), lambda i: (0, 0)),
            pl.BlockSpec((B, 1), lambda i: (0, 0)),
            pl.BlockSpec(memory_space=pl.ANY),
        ],
        out_specs=pl.BlockSpec((B, 128), lambda i: (0, 0)),
        out_shape=jax.ShapeDtypeStruct((B, 128), jnp.float32),
        scratch_shapes=[
            pltpu.VMEM((NBUF_, CB_, D), jnp.float32),
            pltpu.SemaphoreType.DMA((NBUF_,)),
        ],
        compiler_params=pltpu.CompilerParams(
            dimension_semantics=("arbitrary",)),
    )(a_aug, labels2, target)

    out = pl.pallas_call(
        _final_body,
        in_specs=[
            pl.BlockSpec(memory_space=pltpu.SMEM),
            pl.BlockSpec((B, 128), lambda: (0, 0)),
            pl.BlockSpec((B, D), lambda: (0, 0)),
            pl.BlockSpec(memory_space=pl.ANY),
        ],
        out_specs=pl.BlockSpec(memory_space=pltpu.SMEM),
        out_shape=jax.ShapeDtypeStruct((1, 1), jnp.float32),
        scratch_shapes=[
            pltpu.VMEM((B, D), jnp.float32),
            pltpu.SemaphoreType.DMA,
        ],
    )(labels, minacc, inputs, target)
    return out[0, 0]
